# baseline (speedup 1.0000x reference)
import functools

import jax
import jax.numpy as jnp
from jax import lax
from jax.experimental import pallas as pl
from jax.experimental.pallas import tpu as pltpu
from jax.experimental.pallas import tpu_sc as plsc

D = 128
V = 257
L = 1024
P_ROWS = 2 * L - 1
FILL = L - 129
NC = 2
NS = 16
ROWS_PER_TILE = L // (NC * NS)
FB = 128
NDMA = 12
RPD = 31
RPS = 70
W = 512
NBUF = 2


def _sc_body(table_hbm, out_hbm, p_sh, fill_v, trow_v, span_v, sem_o):
    c = lax.axis_index("c")
    s = lax.axis_index("s")

    @pl.when(s == 0)
    def _():
        pltpu.sync_copy(table_hbm, p_sh.at[pl.ds(FILL, V)])

    def _build_fill(edge_row):
        pltpu.sync_copy(table_hbm.at[pl.ds(edge_row, 1)], trow_v)

        def rep(r, carry):
            for k in range(D // 16):
                fill_v[r, pl.ds(k * 16, 16)] = trow_v[0, pl.ds(k * 16, 16)]
            return carry

        lax.fori_loop(0, FB, rep, 0)

    @pl.when(s == 1)
    def _():
        _build_fill(0)
        pltpu.sync_copy(fill_v.at[pl.ds(0, FILL % FB)],
                        p_sh.at[pl.ds(0, FILL % FB)])
        for b in range(FILL // FB):
            pltpu.sync_copy(fill_v, p_sh.at[pl.ds(FILL % FB + b * FB, FB)])

    @pl.when(s == 2)
    def _():
        _build_fill(V - 1)
        for b in range(FILL // FB):
            pltpu.sync_copy(fill_v, p_sh.at[pl.ds(FILL + V + b * FB, FB)])
        pltpu.sync_copy(fill_v.at[pl.ds(0, FILL % FB)],
                        p_sh.at[pl.ds(P_ROWS - FILL % FB, FILL % FB)])

    plsc.subcore_barrier()

    @pl.when(s < NDMA)
    def _():
        i0 = c * 512 + RPD * s
        inflight = []
        for k in range(RPD):
            i = i0 + k
            if len(inflight) >= NBUF:
                inflight.pop(0).wait()
            inflight.append(
                pltpu.async_copy(p_sh.at[pl.ds(L - 1 - i, L)],
                                 out_hbm.at[i], sem_o))
        for cp in inflight:
            cp.wait()

    @pl.when(s >= NDMA)
    def _():
        p = (s - NDMA) // 2
        g = (s - NDMA) % 2
        i0 = c * 512 + NDMA * RPD + RPS * p
        span0 = L - i0 - RPS + W * g
        pltpu.sync_copy(p_sh.at[pl.ds(span0, RPS + W)], span_v)
        inflight = []
        for q in range(RPS):
            if len(inflight) >= 2 * NBUF:
                inflight.pop(0).wait()
            inflight.append(
                pltpu.async_copy(span_v.at[pl.ds(RPS - 1 - q, W)],
                                 out_hbm.at[i0 + q, pl.ds(W * g, W)], sem_o))
        for cp in inflight:
            cp.wait()


@functools.partial(
    pl.kernel,
    out_type=jax.ShapeDtypeStruct((L, L, D), jnp.float32),
    mesh=plsc.VectorSubcoreMesh(core_axis_name="c", subcore_axis_name="s"),
    scratch_types=[
        pltpu.VMEM_SHARED((P_ROWS, D), jnp.float32),
        pltpu.VMEM((FB, D), jnp.float32),
        pltpu.VMEM((1, D), jnp.float32),
        pltpu.VMEM((RPS + W, D), jnp.float32),
        pltpu.SemaphoreType.DMA,
    ],
)
def _rel_pos_sc(table_hbm, out_hbm, p_sh, fill_v, trow_v, span_v, sem_o):
    _sc_body(table_hbm, out_hbm, p_sh, fill_v, trow_v, span_v, sem_o)


def kernel(embeddings_table, length):
    return _rel_pos_sc(embeddings_table)


# speedup vs baseline: 1.0740x; 1.0740x over previous
import functools

import jax
import jax.numpy as jnp
from jax import lax
from jax.experimental import pallas as pl
from jax.experimental.pallas import tpu as pltpu
from jax.experimental.pallas import tpu_sc as plsc

D = 128
V = 257
L = 1024
P_ROWS = 2 * L - 1
FILL = L - 129
NC = 2
NS = 16
ROWS_PER_TILE = L // (NC * NS)
FB = 128
NDMA = 12
RPD = 33
RPS = 58
W = 512
NBUF = 2


def _sc_body(table_hbm, out_hbm, p_sh, fill_v, trow_v, span_v, sem_o):
    c = lax.axis_index("c")
    s = lax.axis_index("s")

    @pl.when(s == 0)
    def _():
        pltpu.sync_copy(table_hbm, p_sh.at[pl.ds(FILL, V)])

    def _build_fill(edge_row):
        pltpu.sync_copy(table_hbm.at[pl.ds(edge_row, 1)], trow_v)

        def rep(r, carry):
            for k in range(D // 16):
                fill_v[r, pl.ds(k * 16, 16)] = trow_v[0, pl.ds(k * 16, 16)]
            return carry

        lax.fori_loop(0, FB, rep, 0)

    @pl.when(s == 1)
    def _():
        _build_fill(0)
        pltpu.sync_copy(fill_v.at[pl.ds(0, FILL % FB)],
                        p_sh.at[pl.ds(0, FILL % FB)])
        for b in range(FILL // FB):
            pltpu.sync_copy(fill_v, p_sh.at[pl.ds(FILL % FB + b * FB, FB)])

    @pl.when(s == 2)
    def _():
        _build_fill(V - 1)
        for b in range(FILL // FB):
            pltpu.sync_copy(fill_v, p_sh.at[pl.ds(FILL + V + b * FB, FB)])
        pltpu.sync_copy(fill_v.at[pl.ds(0, FILL % FB)],
                        p_sh.at[pl.ds(P_ROWS - FILL % FB, FILL % FB)])

    plsc.subcore_barrier()

    @pl.when(s < NDMA)
    def _():
        i0 = c * 512 + RPD * s
        inflight = []
        for k in range(RPD):
            i = i0 + k
            if len(inflight) >= NBUF:
                inflight.pop(0).wait()
            inflight.append(
                pltpu.async_copy(p_sh.at[pl.ds(L - 1 - i, L)],
                                 out_hbm.at[i], sem_o))
        for cp in inflight:
            cp.wait()

    @pl.when(s >= NDMA)
    def _():
        p = (s - NDMA) // 2
        g = (s - NDMA) % 2
        i0 = c * 512 + NDMA * RPD + RPS * p
        span0 = L - i0 - RPS + W * g
        pltpu.sync_copy(p_sh.at[pl.ds(span0, RPS + W)], span_v)
        inflight = []
        for q in range(RPS):
            if len(inflight) >= 2 * NBUF:
                inflight.pop(0).wait()
            inflight.append(
                pltpu.async_copy(span_v.at[pl.ds(RPS - 1 - q, W)],
                                 out_hbm.at[i0 + q, pl.ds(W * g, W)], sem_o))
        for cp in inflight:
            cp.wait()


@functools.partial(
    pl.kernel,
    out_type=jax.ShapeDtypeStruct((L, L, D), jnp.float32),
    mesh=plsc.VectorSubcoreMesh(core_axis_name="c", subcore_axis_name="s"),
    scratch_types=[
        pltpu.VMEM_SHARED((P_ROWS, D), jnp.float32),
        pltpu.VMEM((FB, D), jnp.float32),
        pltpu.VMEM((1, D), jnp.float32),
        pltpu.VMEM((RPS + W, D), jnp.float32),
        pltpu.SemaphoreType.DMA,
    ],
)
def _rel_pos_sc(table_hbm, out_hbm, p_sh, fill_v, trow_v, span_v, sem_o):
    _sc_body(table_hbm, out_hbm, p_sh, fill_v, trow_v, span_v, sem_o)


def kernel(embeddings_table, length):
    return _rel_pos_sc(embeddings_table)


# speedup vs baseline: 1.2060x; 1.1229x over previous
import functools

import jax
import jax.numpy as jnp
from jax import lax
from jax.experimental import pallas as pl
from jax.experimental.pallas import tpu as pltpu
from jax.experimental.pallas import tpu_sc as plsc

D = 128
V = 257
L = 1024
P_ROWS = 2 * L - 1
FILL = L - 129
NC = 2
NS = 16
ROWS_PER_TILE = L // (NC * NS)
FB = 128
NDMA = 10
RPD = 38
RPS = 44
W = 512
NBUF = 2


def _sc_body(table_hbm, out_hbm, p_sh, fill_v, trow_v, span_v, sem_o):
    c = lax.axis_index("c")
    s = lax.axis_index("s")

    @pl.when(s == 0)
    def _():
        pltpu.sync_copy(table_hbm, p_sh.at[pl.ds(FILL, V)])

    def _build_fill(edge_row):
        pltpu.sync_copy(table_hbm.at[pl.ds(edge_row, 1)], trow_v)

        def rep(r, carry):
            for k in range(D // 16):
                fill_v[r, pl.ds(k * 16, 16)] = trow_v[0, pl.ds(k * 16, 16)]
            return carry

        lax.fori_loop(0, FB, rep, 0)

    @pl.when(s == 1)
    def _():
        _build_fill(0)
        pltpu.sync_copy(fill_v.at[pl.ds(0, FILL % FB)],
                        p_sh.at[pl.ds(0, FILL % FB)])
        for b in range(FILL // FB):
            pltpu.sync_copy(fill_v, p_sh.at[pl.ds(FILL % FB + b * FB, FB)])

    @pl.when(s == 2)
    def _():
        _build_fill(V - 1)
        for b in range(FILL // FB):
            pltpu.sync_copy(fill_v, p_sh.at[pl.ds(FILL + V + b * FB, FB)])
        pltpu.sync_copy(fill_v.at[pl.ds(0, FILL % FB)],
                        p_sh.at[pl.ds(P_ROWS - FILL % FB, FILL % FB)])

    plsc.subcore_barrier()

    @pl.when(s < NDMA)
    def _():
        i0 = c * 512 + RPD * s
        inflight = []
        for k in range(RPD):
            i = i0 + k
            if len(inflight) >= NBUF:
                inflight.pop(0).wait()
            inflight.append(
                pltpu.async_copy(p_sh.at[pl.ds(L - 1 - i, L)],
                                 out_hbm.at[i], sem_o))
        for cp in inflight:
            cp.wait()

    @pl.when(s >= NDMA)
    def _():
        p = (s - NDMA) // 2
        g = (s - NDMA) % 2
        i0 = c * 512 + NDMA * RPD + RPS * p
        span0 = L - i0 - RPS + W * g
        pltpu.sync_copy(p_sh.at[pl.ds(span0, RPS + W)], span_v)
        inflight = []
        for q in range(RPS):
            if len(inflight) >= 2 * NBUF:
                inflight.pop(0).wait()
            inflight.append(
                pltpu.async_copy(span_v.at[pl.ds(RPS - 1 - q, W)],
                                 out_hbm.at[i0 + q, pl.ds(W * g, W)], sem_o))
        for cp in inflight:
            cp.wait()


@functools.partial(
    pl.kernel,
    out_type=jax.ShapeDtypeStruct((L, L, D), jnp.float32),
    mesh=plsc.VectorSubcoreMesh(core_axis_name="c", subcore_axis_name="s"),
    scratch_types=[
        pltpu.VMEM_SHARED((P_ROWS, D), jnp.float32),
        pltpu.VMEM((FB, D), jnp.float32),
        pltpu.VMEM((1, D), jnp.float32),
        pltpu.VMEM((RPS + W, D), jnp.float32),
        pltpu.SemaphoreType.DMA,
    ],
)
def _rel_pos_sc(table_hbm, out_hbm, p_sh, fill_v, trow_v, span_v, sem_o):
    _sc_body(table_hbm, out_hbm, p_sh, fill_v, trow_v, span_v, sem_o)


def kernel(embeddings_table, length):
    return _rel_pos_sc(embeddings_table)


# speedup vs baseline: 1.2595x; 1.0443x over previous
import functools

import jax
import jax.numpy as jnp
from jax import lax
from jax.experimental import pallas as pl
from jax.experimental.pallas import tpu as pltpu
from jax.experimental.pallas import tpu_sc as plsc

D = 128
V = 257
L = 1024
P_ROWS = 2 * L - 1
FILL = L - 129
NC = 2
NS = 16
ROWS_PER_TILE = L // (NC * NS)
FB = 128
NDMA = 8
RPD = 46
RPS = 36
W = 512
NBUF = 2


def _sc_body(table_hbm, out_hbm, p_sh, fill_v, trow_v, span_v, sem_o):
    c = lax.axis_index("c")
    s = lax.axis_index("s")

    @pl.when(s == 0)
    def _():
        pltpu.sync_copy(table_hbm, p_sh.at[pl.ds(FILL, V)])

    def _build_fill(edge_row):
        pltpu.sync_copy(table_hbm.at[pl.ds(edge_row, 1)], trow_v)

        def rep(r, carry):
            for k in range(D // 16):
                fill_v[r, pl.ds(k * 16, 16)] = trow_v[0, pl.ds(k * 16, 16)]
            return carry

        lax.fori_loop(0, FB, rep, 0)

    @pl.when(s == 1)
    def _():
        _build_fill(0)
        pltpu.sync_copy(fill_v.at[pl.ds(0, FILL % FB)],
                        p_sh.at[pl.ds(0, FILL % FB)])
        for b in range(FILL // FB):
            pltpu.sync_copy(fill_v, p_sh.at[pl.ds(FILL % FB + b * FB, FB)])

    @pl.when(s == 2)
    def _():
        _build_fill(V - 1)
        for b in range(FILL // FB):
            pltpu.sync_copy(fill_v, p_sh.at[pl.ds(FILL + V + b * FB, FB)])
        pltpu.sync_copy(fill_v.at[pl.ds(0, FILL % FB)],
                        p_sh.at[pl.ds(P_ROWS - FILL % FB, FILL % FB)])

    plsc.subcore_barrier()

    @pl.when(s < NDMA)
    def _():
        i0 = c * 512 + RPD * s
        inflight = []
        for k in range(RPD):
            i = i0 + k
            if len(inflight) >= NBUF:
                inflight.pop(0).wait()
            inflight.append(
                pltpu.async_copy(p_sh.at[pl.ds(L - 1 - i, L)],
                                 out_hbm.at[i], sem_o))
        for cp in inflight:
            cp.wait()

    @pl.when(s >= NDMA)
    def _():
        p = (s - NDMA) // 2
        g = (s - NDMA) % 2
        i0 = c * 512 + NDMA * RPD + RPS * p
        span0 = L - i0 - RPS + W * g
        pltpu.sync_copy(p_sh.at[pl.ds(span0, RPS + W)], span_v)
        inflight = []
        for q in range(RPS):
            if len(inflight) >= 2 * NBUF:
                inflight.pop(0).wait()
            inflight.append(
                pltpu.async_copy(span_v.at[pl.ds(RPS - 1 - q, W)],
                                 out_hbm.at[i0 + q, pl.ds(W * g, W)], sem_o))
        for cp in inflight:
            cp.wait()


@functools.partial(
    pl.kernel,
    out_type=jax.ShapeDtypeStruct((L, L, D), jnp.float32),
    mesh=plsc.VectorSubcoreMesh(core_axis_name="c", subcore_axis_name="s"),
    scratch_types=[
        pltpu.VMEM_SHARED((P_ROWS, D), jnp.float32),
        pltpu.VMEM((FB, D), jnp.float32),
        pltpu.VMEM((1, D), jnp.float32),
        pltpu.VMEM((RPS + W, D), jnp.float32),
        pltpu.SemaphoreType.DMA,
    ],
)
def _rel_pos_sc(table_hbm, out_hbm, p_sh, fill_v, trow_v, span_v, sem_o):
    _sc_body(table_hbm, out_hbm, p_sh, fill_v, trow_v, span_v, sem_o)


def kernel(embeddings_table, length):
    return _rel_pos_sc(embeddings_table)
